# baseline (speedup 1.0000x reference)
import functools

import jax
import jax.numpy as jnp
from jax import lax
from jax.experimental import pallas as pl
from jax.experimental.pallas import tpu as pltpu
from jax.experimental.pallas import tpu_sc as plsc

Q = 8
D = 768
R = 4
KCOEF = 0.1
BK = 1000
NBH = 25
NS = 4
_BIG_I32 = 2147483647


def _block_top4(qn, x, ml, lf, base):
    rawT = lax.dot_general(
        x, qn, (((1,), (1,)), ((), ())),
        preferred_element_type=jnp.float32,
    )
    xss = jnp.sum(x * x, axis=1, keepdims=True)
    sem = rawT * lax.rsqrt(jnp.maximum(xss, 1e-16))

    rel = jnp.abs(ml - lf) / jnp.maximum(ml, lf)
    s = sem * jnp.exp(-rel * KCOEF)

    giota = lax.broadcasted_iota(jnp.int32, (BK, Q), 0) + base
    bvals, bidx = [], []
    for _ in range(R):
        m = jnp.max(s, axis=0, keepdims=True)
        am = jnp.min(jnp.where(s == m, giota, _BIG_I32), axis=0,
                     keepdims=True)
        bvals.append(m)
        bidx.append(am)
        s = jnp.where(giota == am, -jnp.inf, s)
    return bvals, bidx


def _score_topk_body(q_ref, xa_ref, xb_ref, xc_ref, xd_ref,
                     mla_ref, mlb_ref, mlc_ref, mld_ref, lf_ref,
                     vals_ref, idx_ref):
    i = pl.program_id(0)

    @pl.when(i == 0)
    def _init():
        vals_ref[...] = jnp.full((R, Q), -jnp.inf, jnp.float32)
        idx_ref[...] = jnp.zeros((R, Q), jnp.int32)

    q = q_ref[...]
    lf = lf_ref[...]

    qss = jnp.sum(q * q, axis=1, keepdims=True)
    qn = q * lax.rsqrt(jnp.maximum(qss, 1e-16))

    cand_v, cand_i = [], []
    for st, (xr, mr) in enumerate(
            [(xa_ref, mla_ref), (xb_ref, mlb_ref),
             (xc_ref, mlc_ref), (xd_ref, mld_ref)]):
        sv, si = _block_top4(qn, xr[...], mr[...], lf, (i + st * NBH) * BK)
        cand_v += sv
        cand_i += si

    cat_v = jnp.concatenate([vals_ref[...]] + cand_v, axis=0)
    cat_i = jnp.concatenate([idx_ref[...]] + cand_i, axis=0)
    pos = lax.broadcasted_iota(jnp.int32, ((NS + 1) * R, Q), 0)
    nv, ni = [], []
    for _ in range(R):
        m = jnp.max(cat_v, axis=0, keepdims=True)
        p = jnp.min(jnp.where(cat_v == m, pos, _BIG_I32), axis=0,
                    keepdims=True)
        sel = pos == p
        nv.append(m)
        ni.append(jnp.sum(jnp.where(sel, cat_i, 0), axis=0, keepdims=True))
        cat_v = jnp.where(sel, -jnp.inf, cat_v)
    vals_ref[...] = jnp.concatenate(nv, axis=0)
    idx_ref[...] = jnp.concatenate(ni, axis=0)


def _score_topk(query, x, ml2, lf, interpret=False):
    return pl.pallas_call(
        _score_topk_body,
        grid=(NBH,),
        in_specs=[
            pl.BlockSpec((Q, D), lambda i: (0, 0)),
            pl.BlockSpec((BK, D), lambda i: (i, 0)),
            pl.BlockSpec((BK, D), lambda i: (i + NBH, 0)),
            pl.BlockSpec((BK, D), lambda i: (i + 2 * NBH, 0)),
            pl.BlockSpec((BK, D), lambda i: (i + 3 * NBH, 0)),
            pl.BlockSpec((BK, 1), lambda i: (i, 0)),
            pl.BlockSpec((BK, 1), lambda i: (i + NBH, 0)),
            pl.BlockSpec((BK, 1), lambda i: (i + 2 * NBH, 0)),
            pl.BlockSpec((BK, 1), lambda i: (i + 3 * NBH, 0)),
            pl.BlockSpec((1, Q), lambda i: (0, 0)),
        ],
        out_specs=[
            pl.BlockSpec((R, Q), lambda i: (0, 0)),
            pl.BlockSpec((R, Q), lambda i: (0, 0)),
        ],
        out_shape=[
            jax.ShapeDtypeStruct((R, Q), jnp.float32),
            jax.ShapeDtypeStruct((R, Q), jnp.int32),
        ],
        interpret=interpret,
    )(query, x, x, x, x, ml2, ml2, ml2, ml2, lf)


@functools.lru_cache(maxsize=1)
def _build_gather_weight():
    mesh = plsc.VectorSubcoreMesh(core_axis_name="c", subcore_axis_name="s")

    @functools.partial(
        pl.kernel,
        mesh=mesh,
        out_type=jax.ShapeDtypeStruct((Q * R, D), jnp.float32),
        scratch_types=[
            pltpu.VMEM((Q * R,), jnp.int32),
            pltpu.VMEM((16,), jnp.float32),
            pltpu.VMEM((Q * R, D), jnp.float32),
            pltpu.VMEM((D,), jnp.float32),
            pltpu.SemaphoreType.DMA,
        ],
    )
    def gather_weight(idx_hbm, scb_hbm, table_hbm, out_hbm,
                      idx_v, scb_v, rows_v, out_v, sem):
        w = lax.axis_index("s") * 2 + lax.axis_index("c")
        pltpu.sync_copy(idx_hbm, idx_v)
        pltpu.async_copy(table_hbm.at[idx_v], rows_v, sem).wait()
        pltpu.sync_copy(scb_hbm.at[w], scb_v)
        score = scb_v[...]
        for j in range(D // 16):
            out_v[pl.ds(j * 16, 16)] = rows_v[w, pl.ds(j * 16, 16)] * score
        pltpu.sync_copy(out_v, out_hbm.at[w])

    return gather_weight


def kernel(query, text_features, lengths, motion_lengths):
    k = text_features.shape[0]
    lf = lengths.astype(jnp.float32).reshape(1, Q)
    ml2 = motion_lengths.astype(jnp.float32).reshape(k, 1)
    vals_t, idx_t = _score_topk(query, text_features, ml2, lf)
    top_vals = vals_t.T
    top_idx = idx_t.T
    score_bcast = jnp.broadcast_to(top_vals.reshape(Q * R, 1), (Q * R, 16))
    weighted = _build_gather_weight()(top_idx.reshape(Q * R),
                                      score_bcast,
                                      text_features)
    return weighted.reshape(Q, R, D), top_idx


# speedup vs baseline: 1.0702x; 1.0702x over previous
import functools

import jax
import jax.numpy as jnp
from jax import lax
from jax.experimental import pallas as pl
from jax.experimental.pallas import tpu as pltpu
from jax.experimental.pallas import tpu_sc as plsc

Q = 8
D = 768
R = 4
KCOEF = 0.1
BK = 2000
NBH = 25
_BIG_I32 = 2147483647


def _block_top4(qn, x, ml, lf, base):
    rawT = lax.dot_general(
        x, qn, (((1,), (1,)), ((), ())),
        preferred_element_type=jnp.float32,
    )
    xss = jnp.sum(x * x, axis=1, keepdims=True)
    sem = rawT * lax.rsqrt(jnp.maximum(xss, 1e-16))

    rel = jnp.abs(ml - lf) / jnp.maximum(ml, lf)
    s = sem * jnp.exp(-rel * KCOEF)

    giota = lax.broadcasted_iota(jnp.int32, (BK, Q), 0) + base
    bvals, bidx = [], []
    for _ in range(R):
        m = jnp.max(s, axis=0, keepdims=True)
        am = jnp.min(jnp.where(s == m, giota, _BIG_I32), axis=0,
                     keepdims=True)
        bvals.append(m)
        bidx.append(am)
        s = jnp.where(giota == am, -jnp.inf, s)
    return bvals, bidx


def _score_topk_body(q_ref, xa_ref, xb_ref, mla_ref, mlb_ref, lf_ref,
                     vals_ref, idx_ref):
    i = pl.program_id(0)

    @pl.when(i == 0)
    def _init():
        vals_ref[...] = jnp.full((R, Q), -jnp.inf, jnp.float32)
        idx_ref[...] = jnp.zeros((R, Q), jnp.int32)

    q = q_ref[...]
    lf = lf_ref[...]

    qss = jnp.sum(q * q, axis=1, keepdims=True)
    qn = q * lax.rsqrt(jnp.maximum(qss, 1e-16))

    av, ai = _block_top4(qn, xa_ref[...], mla_ref[...], lf, i * BK)
    bv, bi = _block_top4(qn, xb_ref[...], mlb_ref[...], lf, (i + NBH) * BK)

    cat_v = jnp.concatenate([vals_ref[...]] + av + bv, axis=0)
    cat_i = jnp.concatenate([idx_ref[...]] + ai + bi, axis=0)
    pos = lax.broadcasted_iota(jnp.int32, (3 * R, Q), 0)
    nv, ni = [], []
    for _ in range(R):
        m = jnp.max(cat_v, axis=0, keepdims=True)
        p = jnp.min(jnp.where(cat_v == m, pos, _BIG_I32), axis=0,
                    keepdims=True)
        sel = pos == p
        nv.append(m)
        ni.append(jnp.sum(jnp.where(sel, cat_i, 0), axis=0, keepdims=True))
        cat_v = jnp.where(sel, -jnp.inf, cat_v)
    vals_ref[...] = jnp.concatenate(nv, axis=0)
    idx_ref[...] = jnp.concatenate(ni, axis=0)


def _score_topk(query, x, ml2, lf, interpret=False):
    return pl.pallas_call(
        _score_topk_body,
        grid=(NBH,),
        in_specs=[
            pl.BlockSpec((Q, D), lambda i: (0, 0)),
            pl.BlockSpec((BK, D), lambda i: (i, 0)),
            pl.BlockSpec((BK, D), lambda i: (i + NBH, 0)),
            pl.BlockSpec((BK, 1), lambda i: (i, 0)),
            pl.BlockSpec((BK, 1), lambda i: (i + NBH, 0)),
            pl.BlockSpec((1, Q), lambda i: (0, 0)),
        ],
        out_specs=[
            pl.BlockSpec((R, Q), lambda i: (0, 0)),
            pl.BlockSpec((R, Q), lambda i: (0, 0)),
        ],
        out_shape=[
            jax.ShapeDtypeStruct((R, Q), jnp.float32),
            jax.ShapeDtypeStruct((R, Q), jnp.int32),
        ],
        interpret=interpret,
    )(query, x, x, ml2, ml2, lf)


@functools.lru_cache(maxsize=1)
def _build_gather_weight():
    mesh = plsc.VectorSubcoreMesh(core_axis_name="c", subcore_axis_name="s")

    @functools.partial(
        pl.kernel,
        mesh=mesh,
        out_type=jax.ShapeDtypeStruct((Q * R, D), jnp.float32),
        scratch_types=[
            pltpu.VMEM((Q * R,), jnp.int32),
            pltpu.VMEM((16,), jnp.float32),
            pltpu.VMEM((Q * R, D), jnp.float32),
            pltpu.VMEM((D,), jnp.float32),
            pltpu.SemaphoreType.DMA,
        ],
    )
    def gather_weight(idx_hbm, scb_hbm, table_hbm, out_hbm,
                      idx_v, scb_v, rows_v, out_v, sem):
        w = lax.axis_index("s") * 2 + lax.axis_index("c")
        pltpu.sync_copy(idx_hbm, idx_v)
        pltpu.async_copy(table_hbm.at[idx_v], rows_v, sem).wait()
        pltpu.sync_copy(scb_hbm.at[w], scb_v)
        score = scb_v[...]
        for j in range(D // 16):
            out_v[pl.ds(j * 16, 16)] = rows_v[w, pl.ds(j * 16, 16)] * score
        pltpu.sync_copy(out_v, out_hbm.at[w])

    return gather_weight


def kernel(query, text_features, lengths, motion_lengths):
    k = text_features.shape[0]
    lf = lengths.astype(jnp.float32).reshape(1, Q)
    ml2 = motion_lengths.astype(jnp.float32).reshape(k, 1)
    vals_t, idx_t = _score_topk(query, text_features, ml2, lf)
    top_vals = vals_t.T
    top_idx = idx_t.T
    score_bcast = jnp.broadcast_to(top_vals.reshape(Q * R, 1), (Q * R, 16))
    weighted = _build_gather_weight()(top_idx.reshape(Q * R),
                                      score_bcast,
                                      text_features)
    return weighted.reshape(Q, R, D), top_idx
